# baseline (speedup 1.0000x reference)
import jax
import jax.numpy as jnp
from jax.experimental import pallas as pl
from jax.experimental.pallas import tpu as pltpu

_TC = 128
_PC = 1024


def _pool_kernel(pf_ref, tb_ref, pbt_ref, w_ref, b_ref,
                 out_ref, rm_ref, pf16_scr, p16_scr):
    pf16_scr[...] = pf_ref[0].astype(jnp.bfloat16)
    tb = tb_ref[0]
    pbt = pbt_ref[0]

    s_row = jax.lax.dot_general(
        w_ref[...], pf16_scr[...], (((1,), (1,)), ((), ())),
        preferred_element_type=jnp.float32) + b_ref[0, 0]
    e_row = jnp.exp(jnp.clip(s_row, -80.0, 80.0))

    t_total = tb.shape[0]
    p_total = pbt.shape[1]
    tc = min(_TC, t_total)
    pc = min(_PC, p_total)
    l_parts = []
    for c in range(t_total // tc):
        sl = slice(c * tc, (c + 1) * tc)
        tb_c = tb[sl, :]
        l_sub = []
        for j in range(p_total // pc):
            pj = slice(j * pc, (j + 1) * pc)
            d0 = pbt[0:1, pj] - tb_c[:, 0:1]
            d1 = pbt[1:2, pj] - tb_c[:, 1:2]
            d2 = tb_c[:, 2:3] - pbt[2:3, pj]
            d3 = tb_c[:, 3:4] - pbt[3:4, pj]
            margin = jnp.minimum(jnp.minimum(d0, d1), jnp.minimum(d2, d3))
            p_cj = jnp.where(margin >= 0.0, 1.0, 0.0) * e_row[:, pj]
            l_sub.append(jnp.sum(p_cj, axis=-1, keepdims=True))
            p16_scr[sl, pj] = p_cj.astype(jnp.bfloat16)
        l_tot = l_sub[0]
        for part in l_sub[1:]:
            l_tot = l_tot + part
        l_parts.append(l_tot)

    l = (l_parts[0] if len(l_parts) == 1
         else jnp.concatenate(l_parts, axis=0))
    acc = jnp.dot(p16_scr[...], pf16_scr[...],
                  preferred_element_type=jnp.float32)
    inv = 1.0 / jnp.where(l > 0.0, l, 1.0)
    out_ref[0] = acc * inv
    rm_ref[0] = jnp.where(l > 0.0, 1.0, 0.0)


def kernel(patch_feats, token_boxes, patch_boxes, token_mask, w_score, b_score):
    B, P, D = patch_feats.shape
    T = token_boxes.shape[1]

    pbt = jnp.swapaxes(patch_boxes, 1, 2)
    invalid = jnp.array([4.0, 4.0, -4.0, -4.0], dtype=jnp.float32)
    tb_adj = jnp.where(token_mask.astype(bool)[:, :, None],
                       token_boxes.astype(jnp.float32), invalid)
    w2 = w_score.reshape(1, D).astype(jnp.bfloat16)
    b2 = b_score.reshape(1, 1).astype(jnp.float32)

    out, rm = pl.pallas_call(
        _pool_kernel,
        grid=(B,),
        in_specs=[
            pl.BlockSpec((1, P, D), lambda b: (b, 0, 0)),
            pl.BlockSpec((1, T, 4), lambda b: (b, 0, 0)),
            pl.BlockSpec((1, 4, P), lambda b: (b, 0, 0)),
            pl.BlockSpec((1, D), lambda b: (0, 0)),
            pl.BlockSpec((1, 1), lambda b: (0, 0)),
        ],
        out_specs=[
            pl.BlockSpec((1, T, D), lambda b: (b, 0, 0)),
            pl.BlockSpec((1, T, 1), lambda b: (b, 0, 0)),
        ],
        out_shape=[
            jax.ShapeDtypeStruct((B, T, D), jnp.float32),
            jax.ShapeDtypeStruct((B, T, 1), jnp.float32),
        ],
        scratch_shapes=[
            pltpu.VMEM((P, D), jnp.bfloat16),
            pltpu.VMEM((T, P), jnp.bfloat16),
        ],
        compiler_params=pltpu.CompilerParams(
            dimension_semantics=("parallel",),
            vmem_limit_bytes=56 * 1024 * 1024,
        ),
    )(patch_feats, tb_adj, pbt, w2, b2)

    return out, rm.reshape(B, T) > 0.0


# speedup vs baseline: 1.0446x; 1.0446x over previous
import jax
import jax.numpy as jnp
from jax.experimental import pallas as pl
from jax.experimental.pallas import tpu as pltpu

_TC = 512


def _pool_kernel(pf_ref, tb_ref, pbt_ref, w_ref, b_ref,
                 out_ref, rm_ref, pf16_scr, p16_scr):
    pf16_scr[...] = pf_ref[0].astype(jnp.bfloat16)
    tb = tb_ref[0]
    pbt = pbt_ref[0]

    s_row = jax.lax.dot_general(
        w_ref[...], pf16_scr[...], (((1,), (1,)), ((), ())),
        preferred_element_type=jnp.float32) + b_ref[0, 0]
    e_row = jnp.exp(jnp.clip(s_row, -80.0, 80.0))

    t_total = tb.shape[0]
    tc = min(_TC, t_total)
    for c in range(t_total // tc):
        sl = slice(c * tc, (c + 1) * tc)
        tb_c = tb[sl, :]
        d0 = pbt[0:1, :] - tb_c[:, 0:1]
        d1 = pbt[1:2, :] - tb_c[:, 1:2]
        d2 = tb_c[:, 2:3] - pbt[2:3, :]
        d3 = tb_c[:, 3:4] - pbt[3:4, :]
        margin = jnp.minimum(jnp.minimum(d0, d1), jnp.minimum(d2, d3))
        p_c = jnp.where(margin >= 0.0, 1.0, 0.0) * e_row
        l_c = jnp.sum(p_c, axis=-1, keepdims=True)
        p16_scr[sl, :] = p_c.astype(jnp.bfloat16)
        acc = jnp.dot(p16_scr[sl, :], pf16_scr[...],
                      preferred_element_type=jnp.float32)
        inv = 1.0 / jnp.where(l_c > 0.0, l_c, 1.0)
        out_ref[0, sl, :] = acc * inv
        rm_ref[0, sl, :] = jnp.where(l_c > 0.0, 1.0, 0.0)


def kernel(patch_feats, token_boxes, patch_boxes, token_mask, w_score, b_score):
    B, P, D = patch_feats.shape
    T = token_boxes.shape[1]

    pbt = jnp.swapaxes(patch_boxes, 1, 2)
    invalid = jnp.array([4.0, 4.0, -4.0, -4.0], dtype=jnp.float32)
    tb_adj = jnp.where(token_mask.astype(bool)[:, :, None],
                       token_boxes.astype(jnp.float32), invalid)
    w2 = w_score.reshape(1, D).astype(jnp.bfloat16)
    b2 = b_score.reshape(1, 1).astype(jnp.float32)

    out, rm = pl.pallas_call(
        _pool_kernel,
        grid=(B,),
        in_specs=[
            pl.BlockSpec((1, P, D), lambda b: (b, 0, 0)),
            pl.BlockSpec((1, T, 4), lambda b: (b, 0, 0)),
            pl.BlockSpec((1, 4, P), lambda b: (b, 0, 0)),
            pl.BlockSpec((1, D), lambda b: (0, 0)),
            pl.BlockSpec((1, 1), lambda b: (0, 0)),
        ],
        out_specs=[
            pl.BlockSpec((1, T, D), lambda b: (b, 0, 0)),
            pl.BlockSpec((1, T, 1), lambda b: (b, 0, 0)),
        ],
        out_shape=[
            jax.ShapeDtypeStruct((B, T, D), jnp.float32),
            jax.ShapeDtypeStruct((B, T, 1), jnp.float32),
        ],
        scratch_shapes=[
            pltpu.VMEM((P, D), jnp.bfloat16),
            pltpu.VMEM((T, P), jnp.bfloat16),
        ],
        compiler_params=pltpu.CompilerParams(
            dimension_semantics=("parallel",),
            vmem_limit_bytes=56 * 1024 * 1024,
        ),
    )(patch_feats, tb_adj, pbt, w2, b2)

    return out, rm.reshape(B, T) > 0.0
